# baseline (device time: 134133 ns/iter reference)
import jax
import jax.numpy as jnp
from jax import lax
from jax.experimental import pallas as pl
from jax.experimental.pallas import tpu as pltpu

N_Z = 4
H = 8
D = 128
SCALE = D ** -0.5
KV_CHUNK = 512


def _partial_body(q_ref, k_ref, v_ref, u_ref, l_ref):
    kvc = pl.program_id(1)

    @pl.when(kvc == 0)
    def _():
        u_ref[...] = jnp.zeros_like(u_ref)
        l_ref[...] = jnp.zeros_like(l_ref)

    for hh in range(H):
        q = q_ref[0, :, hh, :]
        k = k_ref[0, :, hh, :]
        v = v_ref[0, :, hh, :]
        s = lax.dot_general(
            q, k, (((1,), (1,)), ((), ())), preferred_element_type=jnp.float32
        )
        p = jnp.exp(s * SCALE)
        l = jnp.sum(p, axis=1, keepdims=True)
        u = lax.dot_general(
            p, v, (((1,), (0,)), ((), ())), preferred_element_type=jnp.float32
        )
        u_ref[0, :, hh, :] += u
        l_ref[0, :, hh, :] += jnp.broadcast_to(l, u.shape)


def _allreduce_body(
    u_ref, l_ref, o_ref, accl_ref, comm_ref, comml_ref,
    send_sems, recv_sems, lsend_sems, lrecv_sems,
):
    my_x = lax.axis_index("x")
    my_y = lax.axis_index("y")
    my_z = lax.axis_index("z")
    left = (my_z - 1) % N_Z
    right = (my_z + 1) % N_Z

    barrier_sem = pltpu.get_barrier_semaphore()
    for nbr in (left, right):
        pl.semaphore_signal(
            barrier_sem,
            inc=1,
            device_id=(my_x, my_y, nbr),
            device_id_type=pl.DeviceIdType.MESH,
        )
    pl.semaphore_wait(barrier_sem, 2)

    o_ref[...] = u_ref[...]
    accl_ref[...] = l_ref[:, :, :, 0:1]
    comm_ref[0] = u_ref[...]
    comml_ref[0] = accl_ref[...]

    for h in range(N_Z - 1):
        rdma = pltpu.make_async_remote_copy(
            src_ref=comm_ref.at[h],
            dst_ref=comm_ref.at[h + 1],
            send_sem=send_sems.at[h],
            recv_sem=recv_sems.at[h + 1],
            device_id=(my_x, my_y, right),
            device_id_type=pl.DeviceIdType.MESH,
        )
        rdma_l = pltpu.make_async_remote_copy(
            src_ref=comml_ref.at[h],
            dst_ref=comml_ref.at[h + 1],
            send_sem=lsend_sems.at[h],
            recv_sem=lrecv_sems.at[h + 1],
            device_id=(my_x, my_y, right),
            device_id_type=pl.DeviceIdType.MESH,
        )
        rdma.start()
        rdma_l.start()
        rdma.wait()
        rdma_l.wait()
        o_ref[...] += comm_ref[h + 1]
        accl_ref[...] += comml_ref[h + 1]

    o_ref[...] = o_ref[...] / jnp.broadcast_to(accl_ref[...], o_ref.shape)


def partial_only(Q, K, V):
    b, sq, h, d = Q.shape
    skv = K.shape[1]
    n_chunks = skv // KV_CHUNK

    return pl.pallas_call(
        _partial_body,
        grid=(b, n_chunks),
        in_specs=[
            pl.BlockSpec((1, sq, h, d), lambda i, c: (i, 0, 0, 0)),
            pl.BlockSpec((1, KV_CHUNK, h, d), lambda i, c: (i, c, 0, 0)),
            pl.BlockSpec((1, KV_CHUNK, h, d), lambda i, c: (i, c, 0, 0)),
        ],
        out_specs=[
            pl.BlockSpec((1, sq, h, d), lambda i, c: (i, 0, 0, 0)),
            pl.BlockSpec((1, sq, h, d), lambda i, c: (i, 0, 0, 0)),
        ],
        out_shape=[
            jax.ShapeDtypeStruct((b, sq, h, d), jnp.float32),
            jax.ShapeDtypeStruct((b, sq, h, d), jnp.float32),
        ],
    )(Q, K, V)


def kernel(Q, K, V):
    b, sq, h, d = Q.shape
    u, l = partial_only(Q, K, V)

    return pl.pallas_call(
        _allreduce_body,
        in_specs=[
            pl.BlockSpec(memory_space=pltpu.VMEM),
            pl.BlockSpec(memory_space=pltpu.VMEM),
        ],
        out_specs=pl.BlockSpec(memory_space=pltpu.VMEM),
        out_shape=jax.ShapeDtypeStruct((b, sq, h, d), jnp.float32),
        scratch_shapes=[
            pltpu.VMEM((b, sq, h, 1), jnp.float32),
            pltpu.VMEM((N_Z, b, sq, h, d), jnp.float32),
            pltpu.VMEM((N_Z, b, sq, h, 1), jnp.float32),
            pltpu.SemaphoreType.DMA((N_Z,)),
            pltpu.SemaphoreType.DMA((N_Z,)),
            pltpu.SemaphoreType.DMA((N_Z,)),
            pltpu.SemaphoreType.DMA((N_Z,)),
        ],
        compiler_params=pltpu.CompilerParams(collective_id=0),
    )(u, l)


# device time: 134002 ns/iter; 1.0010x vs baseline; 1.0010x over previous
import jax
import jax.numpy as jnp
from jax import lax
from jax.experimental import pallas as pl
from jax.experimental.pallas import tpu as pltpu

N_Z = 4
H = 8
D = 128
SCALE = D ** -0.5
KV_CHUNK = 512


def _partial_body(q_ref, k_ref, v_ref, u_ref, l_ref):
    kvc = pl.program_id(1)

    @pl.when(kvc == 0)
    def _():
        u_ref[...] = jnp.zeros_like(u_ref)
        l_ref[...] = jnp.zeros_like(l_ref)

    for hh in range(H):
        q = q_ref[0, :, hh, :]
        k = k_ref[0, :, hh, :]
        v = v_ref[0, :, hh, :]
        s = lax.dot_general(
            q, k, (((1,), (1,)), ((), ())), preferred_element_type=jnp.float32
        )
        p = jnp.exp(s * SCALE)
        l = jnp.sum(p, axis=1, keepdims=True)
        u = lax.dot_general(
            p, v, (((1,), (0,)), ((), ())), preferred_element_type=jnp.float32
        )
        u_ref[0, :, hh, :] += u
        l_ref[0, :, hh, :] += jnp.broadcast_to(l, u.shape)


def _allreduce_body(
    u_ref, l_ref, o_ref, accl_ref, comm_ref, send_sems, recv_sems
):
    my_x = lax.axis_index("x")
    my_y = lax.axis_index("y")
    my_z = lax.axis_index("z")
    left = (my_z - 1) % N_Z
    right = (my_z + 1) % N_Z

    barrier_sem = pltpu.get_barrier_semaphore()
    for nbr in (left, right):
        pl.semaphore_signal(
            barrier_sem,
            inc=1,
            device_id=(my_x, my_y, nbr),
            device_id_type=pl.DeviceIdType.MESH,
        )
    pl.semaphore_wait(barrier_sem, 2)

    o_ref[...] = u_ref[...]
    accl_ref[...] = l_ref[:, :, :, 0:1]
    comm_ref[0, :, :, :, 0:128] = u_ref[...]
    comm_ref[0, :, :, :, 128:129] = accl_ref[...]

    for h in range(N_Z - 1):
        rdma = pltpu.make_async_remote_copy(
            src_ref=comm_ref.at[h],
            dst_ref=comm_ref.at[h + 1],
            send_sem=send_sems.at[h],
            recv_sem=recv_sems.at[h + 1],
            device_id=(my_x, my_y, right),
            device_id_type=pl.DeviceIdType.MESH,
        )
        rdma.start()
        rdma.wait()
        o_ref[...] += comm_ref[h + 1, :, :, :, 0:128]
        accl_ref[...] += comm_ref[h + 1, :, :, :, 128:129]

    o_ref[...] = o_ref[...] / jnp.broadcast_to(accl_ref[...], o_ref.shape)


def partial_only(Q, K, V):
    b, sq, h, d = Q.shape
    skv = K.shape[1]
    n_chunks = skv // KV_CHUNK

    return pl.pallas_call(
        _partial_body,
        grid=(b, n_chunks),
        in_specs=[
            pl.BlockSpec((1, sq, h, d), lambda i, c: (i, 0, 0, 0)),
            pl.BlockSpec((1, KV_CHUNK, h, d), lambda i, c: (i, c, 0, 0)),
            pl.BlockSpec((1, KV_CHUNK, h, d), lambda i, c: (i, c, 0, 0)),
        ],
        out_specs=[
            pl.BlockSpec((1, sq, h, d), lambda i, c: (i, 0, 0, 0)),
            pl.BlockSpec((1, sq, h, d), lambda i, c: (i, 0, 0, 0)),
        ],
        out_shape=[
            jax.ShapeDtypeStruct((b, sq, h, d), jnp.float32),
            jax.ShapeDtypeStruct((b, sq, h, d), jnp.float32),
        ],
    )(Q, K, V)


def kernel(Q, K, V):
    b, sq, h, d = Q.shape
    u, l = partial_only(Q, K, V)

    return pl.pallas_call(
        _allreduce_body,
        in_specs=[
            pl.BlockSpec(memory_space=pltpu.VMEM),
            pl.BlockSpec(memory_space=pltpu.VMEM),
        ],
        out_specs=pl.BlockSpec(memory_space=pltpu.VMEM),
        out_shape=jax.ShapeDtypeStruct((b, sq, h, d), jnp.float32),
        scratch_shapes=[
            pltpu.VMEM((b, sq, h, 1), jnp.float32),
            pltpu.VMEM((N_Z, b, sq, h, d + 1), jnp.float32),
            pltpu.SemaphoreType.DMA((N_Z,)),
            pltpu.SemaphoreType.DMA((N_Z,)),
        ],
        compiler_params=pltpu.CompilerParams(collective_id=0),
    )(u, l)


# device time: 118089 ns/iter; 1.1359x vs baseline; 1.1348x over previous
import jax
import jax.numpy as jnp
from jax import lax
from jax.experimental import pallas as pl
from jax.experimental.pallas import tpu as pltpu

N_Z = 4
H = 8
D = 128
SCALE = D ** -0.5
KV_CHUNK = 512


def _partial_body(q_ref, k_ref, v_ref, u_ref, l_ref):
    kvc = pl.program_id(1)

    @pl.when(kvc == 0)
    def _():
        u_ref[...] = jnp.zeros_like(u_ref)
        l_ref[...] = jnp.zeros_like(l_ref)

    for hh in range(H):
        q = q_ref[0, :, hh, :]
        k = k_ref[0, :, hh, :]
        v = v_ref[0, :, hh, :]
        s = lax.dot_general(
            q, k, (((1,), (1,)), ((), ())), preferred_element_type=jnp.float32
        )
        p = jnp.exp(s * SCALE)
        l = jnp.sum(p, axis=1, keepdims=True)
        u = lax.dot_general(
            p, v, (((1,), (0,)), ((), ())), preferred_element_type=jnp.float32
        )
        u_ref[0, :, hh, :] += u
        l_ref[0, :, hh:hh + 1] += l


def _allreduce_body(
    u_ref, l_ref, o_ref, accl_ref, commu_ref, comml_ref,
    usend_sems, urecv_sems, lsend_sems, lrecv_sems,
):
    my_x = lax.axis_index("x")
    my_y = lax.axis_index("y")
    my_z = lax.axis_index("z")
    left = (my_z - 1) % N_Z
    right = (my_z + 1) % N_Z

    barrier_sem = pltpu.get_barrier_semaphore()
    for nbr in (left, right):
        pl.semaphore_signal(
            barrier_sem,
            inc=1,
            device_id=(my_x, my_y, nbr),
            device_id_type=pl.DeviceIdType.MESH,
        )
    pl.semaphore_wait(barrier_sem, 2)

    o_ref[...] = u_ref[...]
    accl_ref[...] = l_ref[...]
    commu_ref[0] = u_ref[...]
    comml_ref[0] = l_ref[...]

    for h in range(N_Z - 1):
        rdma_u = pltpu.make_async_remote_copy(
            src_ref=commu_ref.at[h],
            dst_ref=commu_ref.at[h + 1],
            send_sem=usend_sems.at[h],
            recv_sem=urecv_sems.at[h + 1],
            device_id=(my_x, my_y, right),
            device_id_type=pl.DeviceIdType.MESH,
        )
        rdma_l = pltpu.make_async_remote_copy(
            src_ref=comml_ref.at[h],
            dst_ref=comml_ref.at[h + 1],
            send_sem=lsend_sems.at[h],
            recv_sem=lrecv_sems.at[h + 1],
            device_id=(my_x, my_y, right),
            device_id_type=pl.DeviceIdType.MESH,
        )
        rdma_u.start()
        rdma_l.start()
        rdma_u.wait()
        rdma_l.wait()
        o_ref[...] += commu_ref[h + 1]
        accl_ref[...] += comml_ref[h + 1]

    for bb in range(o_ref.shape[0]):
        for hh in range(H):
            lcol = accl_ref[bb, :, hh:hh + 1]
            o_ref[bb, :, hh, :] = o_ref[bb, :, hh, :] / lcol


def partial_only(Q, K, V):
    b, sq, h, d = Q.shape
    skv = K.shape[1]
    n_chunks = skv // KV_CHUNK

    return pl.pallas_call(
        _partial_body,
        grid=(b, n_chunks),
        in_specs=[
            pl.BlockSpec((1, sq, h, d), lambda i, c: (i, 0, 0, 0)),
            pl.BlockSpec((1, KV_CHUNK, h, d), lambda i, c: (i, c, 0, 0)),
            pl.BlockSpec((1, KV_CHUNK, h, d), lambda i, c: (i, c, 0, 0)),
        ],
        out_specs=[
            pl.BlockSpec((1, sq, h, d), lambda i, c: (i, 0, 0, 0)),
            pl.BlockSpec((1, sq, d), lambda i, c: (i, 0, 0)),
        ],
        out_shape=[
            jax.ShapeDtypeStruct((b, sq, h, d), jnp.float32),
            jax.ShapeDtypeStruct((b, sq, d), jnp.float32),
        ],
    )(Q, K, V)


def kernel(Q, K, V):
    b, sq, h, d = Q.shape
    u, l = partial_only(Q, K, V)

    return pl.pallas_call(
        _allreduce_body,
        in_specs=[
            pl.BlockSpec(memory_space=pltpu.VMEM),
            pl.BlockSpec(memory_space=pltpu.VMEM),
        ],
        out_specs=pl.BlockSpec(memory_space=pltpu.VMEM),
        out_shape=jax.ShapeDtypeStruct((b, sq, h, d), jnp.float32),
        scratch_shapes=[
            pltpu.VMEM((b, sq, d), jnp.float32),
            pltpu.VMEM((N_Z, b, sq, h, d), jnp.float32),
            pltpu.VMEM((N_Z, b, sq, d), jnp.float32),
            pltpu.SemaphoreType.DMA((N_Z,)),
            pltpu.SemaphoreType.DMA((N_Z,)),
            pltpu.SemaphoreType.DMA((N_Z,)),
            pltpu.SemaphoreType.DMA((N_Z,)),
        ],
        compiler_params=pltpu.CompilerParams(collective_id=0),
    )(u, l)
